# baseline (speedup 1.0000x reference)
import functools

import jax
import jax.numpy as jnp
from jax import lax
from jax.experimental import pallas as pl
from jax.experimental.pallas import tpu as pltpu
from jax.experimental.pallas import tpu_sc as plsc

V = 1000000
D = 32
B = 16384

_info = plsc.get_sparse_core_info()
_NC, _NS = _info.num_cores, _info.num_subcores
_NW = _NC * _NS
_BPW = B // _NW
_NB = 16
_NG = _BPW // _NB

_mesh = plsc.VectorSubcoreMesh(core_axis_name="c", subcore_axis_name="s")


@functools.partial(
    pl.kernel,
    mesh=_mesh,
    out_type=[
        jax.ShapeDtypeStruct((B, D), jnp.float32),
        jax.ShapeDtypeStruct((B, D), jnp.float32),
        jax.ShapeDtypeStruct((B, D), jnp.float32),
    ],
    scratch_types=[
        pltpu.VMEM((1, _BPW), jnp.int32),
        pltpu.SemaphoreType.DMA,
        pltpu.SemaphoreType.DMA,
        pltpu.SemaphoreType.DMA,
    ],
    compiler_params=pltpu.CompilerParams(needs_layout_passes=False),
)
def _gather3(idx_hbm, wh_hbm, wd_hbm, ww_hbm, oh_hbm, od_hbm, ow_hbm,
             idx_v, semh, semd, semw):
    wid = lax.axis_index("s") * _NC + lax.axis_index("c")
    base = wid * _BPW
    pltpu.sync_copy(idx_hbm.at[wid], idx_v)
    tabs = (wh_hbm, wd_hbm, ww_hbm)
    outs = (oh_hbm, od_hbm, ow_hbm)
    sems = (semh, semd, semw)

    def fire(g):
        vec = idx_v[0, pl.ds(g * _NB, _NB)]
        gb = base + g * _NB
        for l in range(_NB):
            rid = vec[l]
            for t in range(3):
                pltpu.async_copy(tabs[t].at[pl.ds(rid, 1)],
                                 outs[t].at[pl.ds(gb + l, 1)], sems[t])

    def drain_group():
        for _ in range(_NB):
            for t in range(3):
                pltpu.make_async_copy(tabs[t].at[pl.ds(0, 1)],
                                      outs[t].at[pl.ds(0, 1)], sems[t]).wait()

    fire(0)

    def body(g, _):
        @pl.when(g < _NG - 1)
        def _():
            fire(g + 1)

        drain_group()
        return 0

    lax.fori_loop(0, _NG, body, 0)


def kernel(index, W_hour, W_day, W_week):
    idx = index.astype(jnp.int32).reshape(_NW, 1, _BPW)
    out = _gather3(idx, W_hour, W_day, W_week)
    return tuple(out)


# speedup vs baseline: 1.8154x; 1.8154x over previous
import functools

import jax
import jax.numpy as jnp
from jax import lax
from jax.experimental import pallas as pl
from jax.experimental.pallas import tpu as pltpu
from jax.experimental.pallas import tpu_sc as plsc

V = 1000000
D = 32
B = 16384

_info = plsc.get_sparse_core_info()
_NC, _NS = _info.num_cores, _info.num_subcores
_NW = _NC * _NS
_BPW = B // _NW
_NB = 16
_NG = _BPW // _NB

_mesh = plsc.VectorSubcoreMesh(core_axis_name="c", subcore_axis_name="s")


@functools.partial(
    pl.kernel,
    mesh=_mesh,
    out_type=[
        jax.ShapeDtypeStruct((B, D), jnp.float32),
        jax.ShapeDtypeStruct((B, D), jnp.float32),
        jax.ShapeDtypeStruct((B, D), jnp.float32),
    ],
    scratch_types=[
        pltpu.VMEM((1, _BPW), jnp.int32),
        pltpu.SMEM((1, _BPW), jnp.int32),
        pltpu.VMEM((_BPW, D), jnp.float32),
        pltpu.SemaphoreType.DMA,
    ],
    compiler_params=pltpu.CompilerParams(needs_layout_passes=False),
)
def _gather3(idx_hbm, wh_hbm, wd_hbm, ww_hbm, oh_hbm, od_hbm, ow_hbm,
             idx_v, idx_s, rows, sem):
    wid = lax.axis_index("s") * _NC + lax.axis_index("c")
    base = wid * _BPW
    pltpu.sync_copy(idx_hbm.at[wid], idx_v)
    tabs = (wh_hbm, wd_hbm, ww_hbm)
    outs = (oh_hbm, od_hbm, ow_hbm)

    def extract(g, _):
        vec = idx_v[0, pl.ds(g * _NB, _NB)]
        for l in range(_NB):
            idx_s[0, g * _NB + l] = vec[l]
        return 0

    lax.fori_loop(0, _NG, extract, 0)

    for t in range(3):
        tab = tabs[t]

        def fire(g, _):
            gb = g * _NB
            for l in range(_NB):
                rid = idx_s[0, gb + l]
                pltpu.async_copy(tab.at[pl.ds(rid, 1)],
                                 rows.at[pl.ds(gb + l, 1)], sem)
            return 0

        def drain(g, _):
            for l in range(_NB):
                pltpu.make_async_copy(tab.at[pl.ds(0, 1)],
                                      rows.at[pl.ds(0, 1)], sem).wait()
            return 0

        lax.fori_loop(0, _NG, fire, 0)
        lax.fori_loop(0, _NG, drain, 0)
        pltpu.sync_copy(rows, outs[t].at[pl.ds(base, _BPW)])


def kernel(index, W_hour, W_day, W_week):
    idx = index.astype(jnp.int32).reshape(_NW, 1, _BPW)
    out = _gather3(idx, W_hour, W_day, W_week)
    return tuple(out)
